# baseline (speedup 1.0000x reference)
import jax
import jax.numpy as jnp
from jax import lax
from jax.experimental import pallas as pl
from jax.experimental.pallas import tpu as pltpu
from jax.experimental.pallas import tpu_sc as plsc

BS, NQ, NT = 8, 8192, 32
MT = 4
L = 16
QSTAGE = 2048
NSTAGE = NQ // QSTAGE


def _cdist_body(pred_ref, anch_ref, tgt_ref, out_ref):
    tx0 = tgt_ref[0, :, 0:1]
    ty0 = tgt_ref[0, :, 1:2]
    tx1 = tgt_ref[0, :, 2:3]
    ty1 = tgt_ref[0, :, 3:4]
    t0 = (tx0 + tx1) * 0.5
    t1 = (ty0 + ty1) * 0.5
    t2 = tx1 - tx0
    t3 = ty1 - ty0

    x0 = pred_ref[0, 0:1, :]
    y0 = pred_ref[0, 1:2, :]
    x1 = pred_ref[0, 2:3, :]
    y1 = pred_ref[0, 3:4, :]
    p0 = (x0 + x1) * 0.5
    p1 = (y0 + y1) * 0.5
    p2 = x1 - x0
    p3 = y1 - y0
    dp = ((jnp.abs(t0 - p0) + jnp.abs(t1 - p1))
          + jnp.abs(t2 - p2)) + jnp.abs(t3 - p3)
    dp_t = jnp.transpose(dp)
    out_ref[0, 0, 0, :, :] = dp_t[:, :L]
    out_ref[0, 0, 1, :, :] = dp_t[:, L:]

    a0 = anch_ref[0:1, :]
    a1 = anch_ref[1:2, :]
    a2 = anch_ref[2:3, :]
    a3 = anch_ref[3:4, :]
    da = ((jnp.abs(t0 - a0) + jnp.abs(t1 - a1))
          + jnp.abs(t2 - a2)) + jnp.abs(t3 - a3)
    da_t = jnp.transpose(da)
    out_ref[0, 1, 0, :, :] = da_t[:, :L]
    out_ref[0, 1, 1, :, :] = da_t[:, L:]


def _topk_body(d_hbm, out_hbm, buf0, buf1, outstage, sem0, sem1):
    c = lax.axis_index("c")
    s = lax.axis_index("s")
    wid = c * 16 + s
    b = wid // 4
    m = (wid // 2) % 2
    th = wid % 2

    bufs = (buf0, buf1)
    sems = (sem0, sem1)
    copies = [None] * NSTAGE
    copies[0] = pltpu.make_async_copy(
        d_hbm.at[wid, pl.ds(0, QSTAGE * L)], buf0, sem0)
    copies[0].start()

    inf = jnp.full((L,), jnp.inf, jnp.float32)
    zeros = jnp.zeros((L,), jnp.int32)
    carry = (inf, inf, inf, inf, zeros, zeros, zeros, zeros)

    for stg in range(NSTAGE):
        buf = bufs[stg % 2]
        copies[stg].wait()
        if stg + 1 < NSTAGE:
            copies[stg + 1] = pltpu.make_async_copy(
                d_hbm.at[wid, pl.ds((stg + 1) * QSTAGE * L, QSTAGE * L)],
                bufs[(stg + 1) % 2], sems[(stg + 1) % 2])
            copies[stg + 1].start()

        def pair_step(g, cr, _buf=buf, _stg=stg):
            v4 = cr[3]
            base = g * (2 * L)
            ds = []
            for i in range(2 * L):
                ds.append(_buf[pl.ds((base + i) * L, L)])
            mns = ds
            while len(mns) > 1:
                mns = [jnp.minimum(mns[2 * i], mns[2 * i + 1])
                       for i in range(len(mns) // 2)]
            need = plsc.all_reduce_population_count(mns[0] < v4)[0] > 0

            def slow(cr):
                v1, v2, v3, v4, i1, i2, i3, i4 = cr
                for i in range(2 * L):
                    d = ds[i]
                    qi = jnp.full((L,), 1, jnp.int32) * (_stg * QSTAGE + base + i)
                    m1 = v1 <= d
                    m2 = v2 <= d
                    m3 = v3 <= d
                    m4 = v4 <= d
                    nv1 = jnp.where(m1, v1, d)
                    ni1 = jnp.where(m1, i1, qi)
                    nv2 = jnp.where(m2, v2, jnp.where(m1, d, v1))
                    ni2 = jnp.where(m2, i2, jnp.where(m1, qi, i1))
                    nv3 = jnp.where(m3, v3, jnp.where(m2, d, v2))
                    ni3 = jnp.where(m3, i3, jnp.where(m2, qi, i2))
                    nv4 = jnp.where(m4, v4, jnp.where(m3, d, v3))
                    ni4 = jnp.where(m4, i4, jnp.where(m3, qi, i3))
                    v1, v2, v3, v4 = nv1, nv2, nv3, nv4
                    i1, i2, i3, i4 = ni1, ni2, ni3, ni4
                return (v1, v2, v3, v4, i1, i2, i3, i4)

            return lax.cond(need, slow, lambda cr: cr, cr)

        carry = lax.fori_loop(0, QSTAGE // (2 * L), pair_step, carry)

    outstage[0, :] = carry[4]
    outstage[1, :] = carry[5]
    outstage[2, :] = carry[6]
    outstage[3, :] = carry[7]
    pltpu.sync_copy(outstage, out_hbm.at[wid])


def kernel(img_size, pred_boxes, anchor_boxes, tgt_boxes):
    bs, nq = pred_boxes.shape[:2]
    nt = tgt_boxes.shape[1]
    pred_t = jnp.transpose(pred_boxes, (0, 2, 1))
    anch_t = jnp.transpose(anchor_boxes, (1, 0))
    tgt_s = tgt_boxes * img_size

    d = pl.pallas_call(
        _cdist_body,
        grid=(bs,),
        in_specs=[
            pl.BlockSpec((1, 4, NQ), lambda i: (i, 0, 0)),
            pl.BlockSpec((4, NQ), lambda i: (0, 0)),
            pl.BlockSpec((1, NT, 4), lambda i: (i, 0, 0)),
        ],
        out_specs=pl.BlockSpec((1, 2, 2, NQ, L), lambda i: (i, 0, 0, 0, 0)),
        out_shape=jax.ShapeDtypeStruct((bs, 2, 2, NQ, L), jnp.float32),
    )(pred_t, anch_t, tgt_s)

    mesh = plsc.VectorSubcoreMesh(core_axis_name="c", subcore_axis_name="s")
    out = pl.kernel(
        _topk_body,
        out_type=jax.ShapeDtypeStruct((4 * bs, MT, L), jnp.int32),
        mesh=mesh,
        scratch_types=[
            pltpu.VMEM((QSTAGE * L,), jnp.float32),
            pltpu.VMEM((QSTAGE * L,), jnp.float32),
            pltpu.VMEM((MT, L), jnp.int32),
            pltpu.SemaphoreType.DMA,
            pltpu.SemaphoreType.DMA,
        ],
        compiler_params=pltpu.CompilerParams(needs_layout_passes=False),
    )(d.reshape(4 * bs, NQ * L))

    I = out.reshape(bs, 2, 2, MT, L).transpose(0, 3, 1, 2, 4).reshape(bs, MT * 2 * nt)
    j_row = jnp.tile(jnp.concatenate([jnp.arange(nt), jnp.arange(nt)]), MT)
    J = jnp.tile(j_row[None, :], (bs, 1))
    return (I, J)


# speedup vs baseline: 1.1437x; 1.1437x over previous
import jax
import jax.numpy as jnp
from jax import lax
from jax.experimental import pallas as pl
from jax.experimental.pallas import tpu as pltpu
from jax.experimental.pallas import tpu_sc as plsc

BS, NQ, NT = 8, 8192, 32
MT = 4
L = 16
QSTAGE = 2048
NSTAGE = NQ // QSTAGE


def _cdist_body(pred_ref, anch_ref, tgt_ref, out_ref):
    tx0 = tgt_ref[0, :, 0:1]
    ty0 = tgt_ref[0, :, 1:2]
    tx1 = tgt_ref[0, :, 2:3]
    ty1 = tgt_ref[0, :, 3:4]
    t0 = (tx0 + tx1) * 0.5
    t1 = (ty0 + ty1) * 0.5
    t2 = tx1 - tx0
    t3 = ty1 - ty0

    x0 = pred_ref[0, 0:1, :]
    y0 = pred_ref[0, 1:2, :]
    x1 = pred_ref[0, 2:3, :]
    y1 = pred_ref[0, 3:4, :]
    p0 = (x0 + x1) * 0.5
    p1 = (y0 + y1) * 0.5
    p2 = x1 - x0
    p3 = y1 - y0
    out_ref[0, 0, :, :] = ((jnp.abs(t0 - p0) + jnp.abs(t1 - p1))
                           + jnp.abs(t2 - p2)) + jnp.abs(t3 - p3)

    a0 = anch_ref[0:1, :]
    a1 = anch_ref[1:2, :]
    a2 = anch_ref[2:3, :]
    a3 = anch_ref[3:4, :]
    out_ref[0, 1, :, :] = ((jnp.abs(t0 - a0) + jnp.abs(t1 - a1))
                           + jnp.abs(t2 - a2)) + jnp.abs(t3 - a3)


def _topk_body(d_hbm, out_hbm, buf0, buf1, outstage, sem0, sem1):
    c = lax.axis_index("c")
    s = lax.axis_index("s")
    wid = c * 16 + s
    b = wid // 4
    m = (wid // 2) % 2
    th = wid % 2

    bufs = (buf0, buf1)
    sems = (sem0, sem1)
    copies = [None] * NSTAGE
    copies[0] = pltpu.make_async_copy(
        d_hbm.at[wid, pl.ds(0, QSTAGE * L)], buf0, sem0)
    copies[0].start()

    inf = jnp.full((L,), jnp.inf, jnp.float32)
    zeros = jnp.zeros((L,), jnp.int32)
    carry = (inf, inf, inf, inf, zeros, zeros, zeros, zeros)

    for stg in range(NSTAGE):
        buf = bufs[stg % 2]
        copies[stg].wait()
        if stg + 1 < NSTAGE:
            copies[stg + 1] = pltpu.make_async_copy(
                d_hbm.at[wid, pl.ds((stg + 1) * QSTAGE * L, QSTAGE * L)],
                bufs[(stg + 1) % 2], sems[(stg + 1) % 2])
            copies[stg + 1].start()

        def pair_step(g, cr, _buf=buf, _stg=stg):
            v4 = cr[3]
            base = g * (2 * L)
            ds = []
            for i in range(2 * L):
                ds.append(_buf[pl.ds((base + i) * L, L)])
            mns = ds
            while len(mns) > 1:
                mns = [jnp.minimum(mns[2 * i], mns[2 * i + 1])
                       for i in range(len(mns) // 2)]
            need = plsc.all_reduce_population_count(mns[0] < v4)[0] > 0

            def slow(cr):
                v1, v2, v3, v4, i1, i2, i3, i4 = cr
                for i in range(2 * L):
                    d = ds[i]
                    qi = jnp.full((L,), 1, jnp.int32) * (_stg * QSTAGE + base + i)
                    m1 = v1 <= d
                    m2 = v2 <= d
                    m3 = v3 <= d
                    m4 = v4 <= d
                    nv1 = jnp.where(m1, v1, d)
                    ni1 = jnp.where(m1, i1, qi)
                    nv2 = jnp.where(m2, v2, jnp.where(m1, d, v1))
                    ni2 = jnp.where(m2, i2, jnp.where(m1, qi, i1))
                    nv3 = jnp.where(m3, v3, jnp.where(m2, d, v2))
                    ni3 = jnp.where(m3, i3, jnp.where(m2, qi, i2))
                    nv4 = jnp.where(m4, v4, jnp.where(m3, d, v3))
                    ni4 = jnp.where(m4, i4, jnp.where(m3, qi, i3))
                    v1, v2, v3, v4 = nv1, nv2, nv3, nv4
                    i1, i2, i3, i4 = ni1, ni2, ni3, ni4
                return (v1, v2, v3, v4, i1, i2, i3, i4)

            return lax.cond(need, slow, lambda cr: cr, cr)

        carry = lax.fori_loop(0, QSTAGE // (2 * L), pair_step, carry)

    outstage[0, :] = carry[4]
    outstage[1, :] = carry[5]
    outstage[2, :] = carry[6]
    outstage[3, :] = carry[7]
    pltpu.sync_copy(outstage, out_hbm.at[wid])


def kernel(img_size, pred_boxes, anchor_boxes, tgt_boxes):
    bs, nq = pred_boxes.shape[:2]
    nt = tgt_boxes.shape[1]
    pred_t = jnp.transpose(pred_boxes, (0, 2, 1))
    anch_t = jnp.transpose(anchor_boxes, (1, 0))
    tgt_s = tgt_boxes * img_size

    d = pl.pallas_call(
        _cdist_body,
        grid=(bs,),
        in_specs=[
            pl.BlockSpec((1, 4, NQ), lambda i: (i, 0, 0)),
            pl.BlockSpec((4, NQ), lambda i: (0, 0)),
            pl.BlockSpec((1, NT, 4), lambda i: (i, 0, 0)),
        ],
        out_specs=pl.BlockSpec((1, 2, NT, NQ), lambda i: (i, 0, 0, 0)),
        out_shape=jax.ShapeDtypeStruct((bs, 2, NT, NQ), jnp.float32),
    )(pred_t, anch_t, tgt_s)
    d = d.reshape(bs, 2, 2, L, NQ).transpose(0, 1, 2, 4, 3)

    mesh = plsc.VectorSubcoreMesh(core_axis_name="c", subcore_axis_name="s")
    out = pl.kernel(
        _topk_body,
        out_type=jax.ShapeDtypeStruct((4 * bs, MT, L), jnp.int32),
        mesh=mesh,
        scratch_types=[
            pltpu.VMEM((QSTAGE * L,), jnp.float32),
            pltpu.VMEM((QSTAGE * L,), jnp.float32),
            pltpu.VMEM((MT, L), jnp.int32),
            pltpu.SemaphoreType.DMA,
            pltpu.SemaphoreType.DMA,
        ],
        compiler_params=pltpu.CompilerParams(needs_layout_passes=False),
    )(d.reshape(4 * bs, NQ * L))

    I = out.reshape(bs, 2, 2, MT, L).transpose(0, 3, 1, 2, 4).reshape(bs, MT * 2 * nt)
    j_row = jnp.tile(jnp.concatenate([jnp.arange(nt), jnp.arange(nt)]), MT)
    J = jnp.tile(j_row[None, :], (bs, 1))
    return (I, J)


# speedup vs baseline: 4.1330x; 3.6136x over previous
import jax
import jax.numpy as jnp
from jax import lax
from jax.experimental import pallas as pl
from jax.experimental.pallas import tpu as pltpu
from jax.experimental.pallas import tpu_sc as plsc

BS, NQ, NT = 8, 8192, 32
MT = 4
L = 16
QSTAGE = 2048
NSTAGE = NQ // QSTAGE


def _cdist_body(pred_ref, anch_ref, tgt_ref, out_ref):
    tx0 = tgt_ref[0, :, 0:1]
    ty0 = tgt_ref[0, :, 1:2]
    tx1 = tgt_ref[0, :, 2:3]
    ty1 = tgt_ref[0, :, 3:4]
    t0 = (tx0 + tx1) * 0.5
    t1 = (ty0 + ty1) * 0.5
    t2 = tx1 - tx0
    t3 = ty1 - ty0

    x0 = pred_ref[0, 0:1, :]
    y0 = pred_ref[0, 1:2, :]
    x1 = pred_ref[0, 2:3, :]
    y1 = pred_ref[0, 3:4, :]
    p0 = (x0 + x1) * 0.5
    p1 = (y0 + y1) * 0.5
    p2 = x1 - x0
    p3 = y1 - y0
    out_ref[0, 0, :, :] = ((jnp.abs(t0 - p0) + jnp.abs(t1 - p1))
                           + jnp.abs(t2 - p2)) + jnp.abs(t3 - p3)

    a0 = anch_ref[0:1, :]
    a1 = anch_ref[1:2, :]
    a2 = anch_ref[2:3, :]
    a3 = anch_ref[3:4, :]
    out_ref[0, 1, :, :] = ((jnp.abs(t0 - a0) + jnp.abs(t1 - a1))
                           + jnp.abs(t2 - a2)) + jnp.abs(t3 - a3)


def _topk_body(d_hbm, out_hbm, buf0, buf1, scr, outstage, sem0, sem1):
    c = lax.axis_index("c")
    s = lax.axis_index("s")
    wid = c * 16 + s
    bm = wid // 2
    th = wid % 2

    rows = pl.ds(th * L, L)
    bufs = (buf0, buf1)
    sems = (sem0, sem1)
    copies = [None] * NSTAGE
    copies[0] = pltpu.make_async_copy(
        d_hbm.at[bm, rows, pl.ds(0, QSTAGE)], buf0, sem0)
    copies[0].start()

    riota16 = jax.lax.iota(jnp.int32, L) * L
    inf = jnp.full((L,), jnp.inf, jnp.float32)
    zeros = jnp.zeros((L,), jnp.int32)
    carry = (inf, inf, inf, inf, zeros, zeros, zeros, zeros)

    for stg in range(NSTAGE):
        buf = bufs[stg % 2]
        copies[stg].wait()
        if stg + 1 < NSTAGE:
            copies[stg + 1] = pltpu.make_async_copy(
                d_hbm.at[bm, rows, pl.ds((stg + 1) * QSTAGE, QSTAGE)],
                bufs[(stg + 1) % 2], sems[(stg + 1) % 2])
            copies[stg + 1].start()

        def pair_step(g, cr, _buf=buf, _stg=stg):
            v4 = cr[3]
            base = g * (2 * L)
            masks = []
            for t in range(L):
                tau = jnp.full((L,), 1, jnp.float32) * v4[t]
                ra = _buf[t, pl.ds(base, L)]
                rb = _buf[t, pl.ds(base + L, L)]
                masks.append(jnp.minimum(ra, rb) < tau)
            mor = masks
            while len(mor) > 1:
                mor = [mor[2 * i] | mor[2 * i + 1]
                       for i in range(len(mor) // 2)]
            need = plsc.all_reduce_population_count(mor[0])[0] > 0

            def slow(cr):
                v1, v2, v3, v4, i1, i2, i3, i4 = cr
                for t in range(L):
                    idx = riota16 + t
                    plsc.store_scatter(scr, [idx], _buf[t, pl.ds(base, L)])
                    plsc.store_scatter(scr, [idx + L * L],
                                       _buf[t, pl.ds(base + L, L)])
                for i in range(2 * L):
                    d = scr[pl.ds(i * L, L)]
                    qi = jnp.full((L,), 1, jnp.int32) * (_stg * QSTAGE + base + i)
                    m1 = v1 <= d
                    m2 = v2 <= d
                    m3 = v3 <= d
                    m4 = v4 <= d
                    nv1 = jnp.where(m1, v1, d)
                    ni1 = jnp.where(m1, i1, qi)
                    nv2 = jnp.where(m2, v2, jnp.where(m1, d, v1))
                    ni2 = jnp.where(m2, i2, jnp.where(m1, qi, i1))
                    nv3 = jnp.where(m3, v3, jnp.where(m2, d, v2))
                    ni3 = jnp.where(m3, i3, jnp.where(m2, qi, i2))
                    nv4 = jnp.where(m4, v4, jnp.where(m3, d, v3))
                    ni4 = jnp.where(m4, i4, jnp.where(m3, qi, i3))
                    v1, v2, v3, v4 = nv1, nv2, nv3, nv4
                    i1, i2, i3, i4 = ni1, ni2, ni3, ni4
                return (v1, v2, v3, v4, i1, i2, i3, i4)

            return lax.cond(need, slow, lambda cr: cr, cr)

        carry = lax.fori_loop(0, QSTAGE // (2 * L), pair_step, carry)

    outstage[0, :] = carry[4]
    outstage[1, :] = carry[5]
    outstage[2, :] = carry[6]
    outstage[3, :] = carry[7]
    pltpu.sync_copy(outstage, out_hbm.at[wid])


def kernel(img_size, pred_boxes, anchor_boxes, tgt_boxes):
    bs, nq = pred_boxes.shape[:2]
    nt = tgt_boxes.shape[1]
    pred_t = jnp.transpose(pred_boxes, (0, 2, 1))
    anch_t = jnp.transpose(anchor_boxes, (1, 0))
    tgt_s = tgt_boxes * img_size

    d = pl.pallas_call(
        _cdist_body,
        grid=(bs,),
        in_specs=[
            pl.BlockSpec((1, 4, NQ), lambda i: (i, 0, 0)),
            pl.BlockSpec((4, NQ), lambda i: (0, 0)),
            pl.BlockSpec((1, NT, 4), lambda i: (i, 0, 0)),
        ],
        out_specs=pl.BlockSpec((1, 2, NT, NQ), lambda i: (i, 0, 0, 0)),
        out_shape=jax.ShapeDtypeStruct((bs, 2, NT, NQ), jnp.float32),
    )(pred_t, anch_t, tgt_s)

    mesh = plsc.VectorSubcoreMesh(core_axis_name="c", subcore_axis_name="s")
    out = pl.kernel(
        _topk_body,
        out_type=jax.ShapeDtypeStruct((4 * bs, MT, L), jnp.int32),
        mesh=mesh,
        scratch_types=[
            pltpu.VMEM((L, QSTAGE), jnp.float32),
            pltpu.VMEM((L, QSTAGE), jnp.float32),
            pltpu.VMEM((2 * L * L,), jnp.float32),
            pltpu.VMEM((MT, L), jnp.int32),
            pltpu.SemaphoreType.DMA,
            pltpu.SemaphoreType.DMA,
        ],
        compiler_params=pltpu.CompilerParams(needs_layout_passes=False),
    )(d.reshape(2 * bs, NT, NQ))

    I = out.reshape(bs, 2, 2, MT, L).transpose(0, 3, 1, 2, 4).reshape(bs, MT * 2 * nt)
    j_row = jnp.tile(jnp.concatenate([jnp.arange(nt), jnp.arange(nt)]), MT)
    J = jnp.tile(j_row[None, :], (bs, 1))
    return (I, J)
